# Initial kernel scaffold; baseline (speedup 1.0000x reference)
import jax
import jax.numpy as jnp
from jax.experimental import pallas as pl


def kernel(x, adj, W1, b1, W2, b2, W3, b3):
    raise NotImplementedError("write your pallas kernel here")


# speedup vs baseline: 1.3337x; 1.3337x over previous
import jax
import jax.numpy as jnp
from jax.experimental import pallas as pl
from jax.experimental.pallas import tpu as pltpu


def _xw_body(x_ref, w_ref, o_ref):
    o_ref[...] = jnp.dot(x_ref[...], w_ref[...],
                         preferred_element_type=jnp.float32)


def _pass1_body(adj_ref, xw_ref, b1_ref, w23_ref, o_ref):
    acc = jnp.dot(adj_ref[...], xw_ref[...],
                  preferred_element_type=jnp.float32)
    h = jnp.maximum(acc + b1_ref[...], 0.0)
    o_ref[...] = jnp.dot(h, w23_ref[...],
                         preferred_element_type=jnp.float32)


def _pass2_body(adj_ref, hw_ref, b23_ref, o1_ref, o2_ref, o3_ref):
    c = o1_ref.shape[1]
    y = jnp.dot(adj_ref[...], hw_ref[...],
                preferred_element_type=jnp.float32) + b23_ref[...]
    y1 = y[:, :c]
    y2 = y[:, c:]
    m1 = jnp.max(y1, axis=1, keepdims=True)
    ls1 = y1 - m1 - jnp.log(jnp.sum(jnp.exp(y1 - m1), axis=1, keepdims=True))
    m2 = jnp.max(y2, axis=1, keepdims=True)
    ls2 = y2 - m2 - jnp.log(jnp.sum(jnp.exp(y2 - m2), axis=1, keepdims=True))
    o1_ref[...] = ls1
    o2_ref[...] = ls2
    o3_ref[...] = jnp.exp(ls1[:, c - 1:c])


def kernel(x, adj, W1, b1, W2, b2, W3, b3):
    N, _ = x.shape
    H = W1.shape[1]
    C = W2.shape[1]
    C2 = W3.shape[1]
    K = C + C2
    BM = 400

    W23 = jnp.concatenate([W2, W3], axis=1)
    b23 = jnp.concatenate([b2, b3])[None, :]
    b1r = b1[None, :]

    xw = pl.pallas_call(
        _xw_body,
        out_shape=jax.ShapeDtypeStruct((N, H), jnp.float32),
    )(x, W1)

    hw = pl.pallas_call(
        _pass1_body,
        grid=(N // BM,),
        in_specs=[
            pl.BlockSpec((BM, N), lambda i: (i, 0)),
            pl.BlockSpec((N, H), lambda i: (0, 0)),
            pl.BlockSpec((1, H), lambda i: (0, 0)),
            pl.BlockSpec((H, K), lambda i: (0, 0)),
        ],
        out_specs=pl.BlockSpec((BM, K), lambda i: (i, 0)),
        out_shape=jax.ShapeDtypeStruct((N, K), jnp.float32),
        compiler_params=pltpu.CompilerParams(
            dimension_semantics=("parallel",)),
    )(adj, xw, b1r, W23)

    out1, out2, out3 = pl.pallas_call(
        _pass2_body,
        grid=(N // BM,),
        in_specs=[
            pl.BlockSpec((BM, N), lambda i: (i, 0)),
            pl.BlockSpec((N, K), lambda i: (0, 0)),
            pl.BlockSpec((1, K), lambda i: (0, 0)),
        ],
        out_specs=[
            pl.BlockSpec((BM, C), lambda i: (i, 0)),
            pl.BlockSpec((BM, C2), lambda i: (i, 0)),
            pl.BlockSpec((BM, 1), lambda i: (i, 0)),
        ],
        out_shape=(
            jax.ShapeDtypeStruct((N, C), jnp.float32),
            jax.ShapeDtypeStruct((N, C2), jnp.float32),
            jax.ShapeDtypeStruct((N, 1), jnp.float32),
        ),
        compiler_params=pltpu.CompilerParams(
            dimension_semantics=("parallel",)),
    )(adj, hw, b23)

    return (out1, out2, out3[:, 0])


# speedup vs baseline: 1.3838x; 1.0376x over previous
import jax
import jax.numpy as jnp
from jax.experimental import pallas as pl
from jax.experimental.pallas import tpu as pltpu


def _fused_body(x_ref, adj_ref, w1_ref, b1_ref, w23_ref, b23_ref,
                o1_ref, o2_ref, o3_ref, xw_ref, hw_ref):
    phase = pl.program_id(0)
    j = pl.program_id(1)
    c = o1_ref.shape[1]
    bm = adj_ref.shape[0]

    @pl.when(jnp.logical_and(phase == 0, j == 0))
    def _():
        xw_ref[...] = jnp.dot(x_ref[...], w1_ref[...],
                              preferred_element_type=jnp.float32)

    @pl.when(phase == 0)
    def _():
        acc = jnp.dot(adj_ref[...], xw_ref[...],
                      preferred_element_type=jnp.float32)
        h = jnp.maximum(acc + b1_ref[...], 0.0)
        hw_ref[pl.ds(j * bm, bm), :] = jnp.dot(
            h, w23_ref[...], preferred_element_type=jnp.float32)

    @pl.when(phase == 1)
    def _():
        y = jnp.dot(adj_ref[...], hw_ref[...],
                    preferred_element_type=jnp.float32) + b23_ref[...]
        y1 = y[:, :c]
        y2 = y[:, c:]
        m1 = jnp.max(y1, axis=1, keepdims=True)
        ls1 = y1 - m1 - jnp.log(
            jnp.sum(jnp.exp(y1 - m1), axis=1, keepdims=True))
        m2 = jnp.max(y2, axis=1, keepdims=True)
        ls2 = y2 - m2 - jnp.log(
            jnp.sum(jnp.exp(y2 - m2), axis=1, keepdims=True))
        o1_ref[...] = ls1
        o2_ref[...] = ls2
        o3_ref[...] = jnp.exp(ls1[:, c - 1:c])


def kernel(x, adj, W1, b1, W2, b2, W3, b3):
    N, Fin = x.shape
    H = W1.shape[1]
    C = W2.shape[1]
    C2 = W3.shape[1]
    K = C + C2
    BM = 400

    W23 = jnp.concatenate([W2, W3], axis=1)
    b23 = jnp.concatenate([b2, b3])[None, :]
    b1r = b1[None, :]

    out1, out2, out3 = pl.pallas_call(
        _fused_body,
        grid=(2, N // BM),
        in_specs=[
            pl.BlockSpec((N, Fin), lambda i, j: (0, 0)),
            pl.BlockSpec((BM, N), lambda i, j: (j, 0)),
            pl.BlockSpec((Fin, H), lambda i, j: (0, 0)),
            pl.BlockSpec((1, H), lambda i, j: (0, 0)),
            pl.BlockSpec((H, K), lambda i, j: (0, 0)),
            pl.BlockSpec((1, K), lambda i, j: (0, 0)),
        ],
        out_specs=[
            pl.BlockSpec((BM, C), lambda i, j: (j, 0)),
            pl.BlockSpec((BM, C2), lambda i, j: (j, 0)),
            pl.BlockSpec((BM, 1), lambda i, j: (j, 0)),
        ],
        out_shape=(
            jax.ShapeDtypeStruct((N, C), jnp.float32),
            jax.ShapeDtypeStruct((N, C2), jnp.float32),
            jax.ShapeDtypeStruct((N, 1), jnp.float32),
        ),
        scratch_shapes=[
            pltpu.VMEM((N, H), jnp.float32),
            pltpu.VMEM((N, K), jnp.float32),
        ],
        compiler_params=pltpu.CompilerParams(
            dimension_semantics=("arbitrary", "arbitrary")),
    )(x, adj, W1, b1r, W23, b23)

    return (out1, out2, out3[:, 0])
